# baseline (speedup 1.0000x reference)
import functools

import jax
import jax.numpy as jnp
from jax.experimental import pallas as pl
from jax.experimental.pallas import tpu as pltpu

_DOT = functools.partial(jnp.dot, preferred_element_type=jnp.float32,
                         precision=jax.lax.Precision.DEFAULT)


def _dense_body(x_ref, w_ref, o_ref):
    o_ref[...] = _DOT(x_ref[...], w_ref[...])


def _dense(x, w):
    return pl.pallas_call(
        _dense_body,
        out_shape=jax.ShapeDtypeStruct((x.shape[0], w.shape[1]), jnp.float32),
    )(x, w)


def _l1_body(adj_ref, s1_ref, b1_ref, w2_ref,
             x1_ref, x2p_ref, s2_ref,
             *, bi, bc, n, npad, nc_ride):
    i = pl.program_id(0)

    if npad > n:
        @pl.when(i == 0)
        def _():
            s2_ref[pl.ds(n, npad - n), :] = jnp.zeros(
                (npad - n, s2_ref.shape[1]), jnp.float32)

    adj_blk = adj_ref[...]
    x1 = jnp.maximum(_DOT(adj_blk, s1_ref[...]) + b1_ref[...], 0.0)
    x1_ref[...] = x1
    s2_ref[pl.ds(i * bi, bi), :] = _DOT(x1, w2_ref[...])

    x2p_ref[...] = jnp.zeros_like(x2p_ref)
    for c in range(nc_ride):
        @pl.when((c + 1) * bc <= i * bi)
        def _(c=c):
            x2p_ref[...] = x2p_ref[...] + _DOT(
                adj_blk[:, c * bc:(c + 1) * bc],
                s2_ref[pl.ds(c * bc, bc), :])


def _l2_body(adj_ref, s2_ref, x2p_ref, b2_ref, x2_ref,
             *, bi, bc, nk, valid_last):
    i = pl.program_id(0)
    k = pl.program_id(1)
    kb = (i * bi) // bc

    @pl.when(k == kb)
    def _():
        x2_ref[...] = x2p_ref[...] + b2_ref[...]

    @pl.when(k >= kb)
    def _():
        blk = adj_ref[...]
        if valid_last != bc:
            col = jax.lax.broadcasted_iota(jnp.int32, blk.shape, 1)
            blk = jnp.where((k < nk - 1) | (col < valid_last), blk, 0.0)
        x2_ref[...] = x2_ref[...] + _DOT(blk, s2_ref[pl.ds(k * bc, bc), :])


def gcn2(x, adj, W1, b1, W2, b2, bi=400, bc=2048):
    n = adj.shape[0]
    h1 = W1.shape[1]
    h2 = W2.shape[1]
    ni = n // bi
    nk = -(-n // bc)
    npad = nk * bc
    valid_last = n - (nk - 1) * bc
    nc_ride = ((ni - 1) * bi) // bc

    s1 = _dense(x, W1)

    x1, x2p, s2p = pl.pallas_call(
        functools.partial(_l1_body, bi=bi, bc=bc, n=n, npad=npad,
                          nc_ride=nc_ride),
        grid=(ni,),
        in_specs=[
            pl.BlockSpec((bi, n), lambda i: (i, 0)),
            pl.BlockSpec((n, h1), lambda i: (0, 0)),
            pl.BlockSpec((1, h1), lambda i: (0, 0)),
            pl.BlockSpec((h1, h2), lambda i: (0, 0)),
        ],
        out_specs=[
            pl.BlockSpec((bi, h1), lambda i: (i, 0)),
            pl.BlockSpec((bi, h2), lambda i: (i, 0)),
            pl.BlockSpec((npad, h2), lambda i: (0, 0)),
        ],
        out_shape=[
            jax.ShapeDtypeStruct((n, h1), jnp.float32),
            jax.ShapeDtypeStruct((n, h2), jnp.float32),
            jax.ShapeDtypeStruct((npad, h2), jnp.float32),
        ],
        compiler_params=pltpu.CompilerParams(
            dimension_semantics=("arbitrary",)
        ),
    )(adj, s1, b1.reshape(1, -1), W2)

    x2 = pl.pallas_call(
        functools.partial(_l2_body, bi=bi, bc=bc, nk=nk,
                          valid_last=valid_last),
        grid=(ni, nk),
        in_specs=[
            pl.BlockSpec((bi, bc),
                         lambda i, k: (i, jnp.maximum(k, (i * bi) // bc))),
            pl.BlockSpec((npad, h2), lambda i, k: (0, 0)),
            pl.BlockSpec((bi, h2), lambda i, k: (i, 0)),
            pl.BlockSpec((1, h2), lambda i, k: (0, 0)),
        ],
        out_specs=pl.BlockSpec((bi, h2), lambda i, k: (i, 0)),
        out_shape=jax.ShapeDtypeStruct((n, h2), jnp.float32),
        compiler_params=pltpu.CompilerParams(
            dimension_semantics=("arbitrary", "arbitrary")
        ),
    )(adj, s2p, x2p, b2.reshape(1, -1))

    return (x1, x2)


def kernel(x, adj, W1, b1, W2, b2):
    return gcn2(x, adj, W1, b1, W2, b2, bi=400, bc=2048)


# speedup vs baseline: 1.0287x; 1.0287x over previous
import functools

import jax
import jax.numpy as jnp
from jax.experimental import pallas as pl
from jax.experimental.pallas import tpu as pltpu


def _mm(a, b):
    return jax.lax.dot_general(
        a, b, (((1,), (0,)), ((), ())),
        preferred_element_type=jnp.float32,
        precision=jax.lax.Precision.DEFAULT)


def _dense_body(x_ref, w_ref, o_ref):
    o_ref[...] = _mm(x_ref[...], w_ref[...]).astype(jnp.bfloat16)


def _dense_bf16(x, w):
    return pl.pallas_call(
        _dense_body,
        out_shape=jax.ShapeDtypeStruct((x.shape[0], w.shape[1]),
                                       jnp.bfloat16),
    )(x, w)


def _l1_body(adj_ref, s1_ref, b1_ref, w2_ref,
             x1_ref, x2p_ref, s_ref,
             *, bi, h1, n, npad):
    i = pl.program_id(0)

    @pl.when(i == 0)
    def _():
        s_ref[...] = jnp.zeros_like(s_ref)
        s_ref[pl.ds(0, n), :h1] = s1_ref[...]

    out = _mm(adj_ref[...], s_ref[pl.ds(0, n), :])
    x1 = jnp.maximum(out[:, :h1] + b1_ref[...], 0.0)
    x1_ref[...] = x1
    x2p_ref[...] = out[:, h1:]
    s_ref[pl.ds(i * bi, bi), h1:] = _mm(x1, w2_ref[...]).astype(jnp.bfloat16)


def _l2_body(adj_ref, s_ref, x2p_ref, b2_ref, x2_ref,
             *, bi, bc, h1, nk, valid_last):
    i = pl.program_id(0)
    k = pl.program_id(1)
    kb = (i * bi) // bc

    @pl.when(k == kb)
    def _():
        x2_ref[...] = x2p_ref[...] + b2_ref[...]

    @pl.when(k >= kb)
    def _():
        blk = adj_ref[...]
        col = jax.lax.broadcasted_iota(jnp.int32, blk.shape, 1)
        left_cut = jnp.where(k == kb, i * bi - k * bc, 0)
        right = jnp.where(k == nk - 1, valid_last, bc)
        blk = jnp.where((col >= left_cut) & (col < right), blk, 0.0)
        x2_ref[...] = x2_ref[...] + _mm(
            blk, s_ref[pl.ds(k * bc, bc), h1:])


def gcn2(x, adj, W1, b1, W2, b2, bi=400, bc=2048):
    n = adj.shape[0]
    h1 = W1.shape[1]
    h2 = W2.shape[1]
    ni = n // bi
    nk = -(-n // bc)
    npad = nk * bc
    valid_last = n - (nk - 1) * bc

    s1 = _dense_bf16(x, W1)
    w2_bf = W2.astype(jnp.bfloat16)

    x1, x2p, s_buf = pl.pallas_call(
        functools.partial(_l1_body, bi=bi, h1=h1, n=n, npad=npad),
        grid=(ni,),
        in_specs=[
            pl.BlockSpec((bi, n), lambda i: (i, 0)),
            pl.BlockSpec((n, h1), lambda i: (0, 0)),
            pl.BlockSpec((1, h1), lambda i: (0, 0)),
            pl.BlockSpec((h1, h2), lambda i: (0, 0)),
        ],
        out_specs=[
            pl.BlockSpec((bi, h1), lambda i: (i, 0)),
            pl.BlockSpec((bi, h2), lambda i: (i, 0)),
            pl.BlockSpec((npad, h1 + h2), lambda i: (0, 0)),
        ],
        out_shape=[
            jax.ShapeDtypeStruct((n, h1), jnp.float32),
            jax.ShapeDtypeStruct((n, h2), jnp.float32),
            jax.ShapeDtypeStruct((npad, h1 + h2), jnp.bfloat16),
        ],
        compiler_params=pltpu.CompilerParams(
            dimension_semantics=("arbitrary",)
        ),
    )(adj, s1, b1.reshape(1, -1), w2_bf)

    x2 = pl.pallas_call(
        functools.partial(_l2_body, bi=bi, bc=bc, h1=h1, nk=nk,
                          valid_last=valid_last),
        grid=(ni, nk),
        in_specs=[
            pl.BlockSpec((bi, bc),
                         lambda i, k: (i, jnp.maximum(k, (i * bi) // bc))),
            pl.BlockSpec((npad, h1 + h2), lambda i, k: (0, 0)),
            pl.BlockSpec((bi, h2), lambda i, k: (i, 0)),
            pl.BlockSpec((1, h2), lambda i, k: (0, 0)),
        ],
        out_specs=pl.BlockSpec((bi, h2), lambda i, k: (i, 0)),
        out_shape=jax.ShapeDtypeStruct((n, h2), jnp.float32),
        compiler_params=pltpu.CompilerParams(
            dimension_semantics=("arbitrary", "arbitrary")
        ),
    )(adj, s_buf, x2p, b2.reshape(1, -1))

    return (x1, x2)


def kernel(x, adj, W1, b1, W2, b2):
    return gcn2(x, adj, W1, b1, W2, b2, bi=400, bc=2048)
